# baseline (speedup 1.0000x reference)
import functools

import jax
import jax.numpy as jnp
from jax import lax
from jax.experimental import pallas as pl
from jax.experimental.pallas import tpu as pltpu
from jax.experimental.pallas import tpu_sc as plsc


_LANES = 16
_SC = 10
_NBUF = 2


def _make_kernel(B, S, D, V):
    info = plsc.get_sparse_core_info()
    NC, NS = info.num_cores, info.num_subcores
    NW = NC * NS
    BT = B // 128
    assert BT == NW and B % 128 == 0
    assert S % (_SC * _NBUF) == 0
    n_pairs = S // (_SC * _NBUF)
    HREG = D // _LANES
    assert HREG * _LANES == D

    mesh = plsc.VectorSubcoreMesh(core_axis_name="c", subcore_axis_name="s")

    @functools.partial(
        pl.kernel,
        mesh=mesh,
        compiler_params=pltpu.CompilerParams(
            use_tc_tiling_on_sc=False, needs_layout_passes=False
        ),
        out_type=jax.ShapeDtypeStruct((S, B, D), jnp.float32),
        scratch_types=[
            pltpu.VMEM((_NBUF, _SC, 128), jnp.int32),
            pltpu.VMEM((_NBUF, _SC, 128, D), jnp.float32),
            pltpu.VMEM((S, D), jnp.float32),
            pltpu.SemaphoreType.DMA,
            pltpu.SemaphoreType.DMA,
        ],
    )
    def k(xt_hbm, item_hbm, pos_hbm, out_hbm, idx_v, rbuf, pos_v, g0, g1):
        gsems = [g0, g1]
        wid = lax.axis_index("s") * NC + lax.axis_index("c")
        b0 = wid * 128

        pltpu.sync_copy(pos_hbm, pos_v)

        def fire_chunk(c_idx, b):
            s0 = c_idx * _SC
            pltpu.sync_copy(
                xt_hbm.at[pl.ds(s0, _SC), pl.ds(b0, 128)], idx_v.at[b]
            )
            for si in range(_SC):
                pltpu.async_copy(
                    item_hbm.at[idx_v.at[b, si]], rbuf.at[b, si], gsems[b]
                )

        def wait_gathers(b):
            for si in range(_SC):
                pltpu.make_async_copy(
                    item_hbm.at[idx_v.at[b, si]], rbuf.at[b, si], gsems[b]
                ).wait()

        def compute(c_idx, b):
            s0 = c_idx * _SC

            def pos_body(si, c1):
                pvs = [
                    pos_v[s0 + si, pl.ds(h * _LANES, _LANES)]
                    for h in range(HREG)
                ]

                def row_body(c, c2):
                    for h in range(HREG):
                        sl = pl.ds(h * _LANES, _LANES)
                        rbuf[b, si, c, sl] = rbuf[b, si, c, sl] + pvs[h]
                    return c2

                lax.fori_loop(0, 128, row_body, 0)
                return c1

            lax.fori_loop(0, _SC, pos_body, 0)

        fire_chunk(0, 0)
        fire_chunk(1, 1)

        def pair_body(t, carry):
            for b in range(_NBUF):
                c_idx = t * _NBUF + b
                wait_gathers(b)
                compute(c_idx, b)
                pltpu.sync_copy(
                    rbuf.at[b],
                    out_hbm.at[pl.ds(c_idx * _SC, _SC), pl.ds(b0, 128)],
                )

                @pl.when(t < n_pairs - 1)
                def _():
                    fire_chunk(c_idx + _NBUF, b)

            return carry

        lax.fori_loop(0, n_pairs, pair_body, 0)

    return k


def kernel(x, item_emb, pos_emb):
    B, S = x.shape
    V, D = item_emb.shape
    xt = x.astype(jnp.int32).T
    out_sbd = _make_kernel(B, S, D, V)(xt, item_emb, pos_emb[:S])
    return jnp.transpose(out_sbd, (1, 0, 2))
